# baseline (speedup 1.0000x reference)
import jax
import jax.numpy as jnp
from jax.experimental import pallas as pl
from jax.experimental.pallas import tpu as pltpu

BATCH = 2
SEQLEN = 4096
D_MODEL = 1024
D_STATE = 128
HEADDIM = 64
D_CONV = 4
D_INNER = 2048
NHEADS = 32
CONV_DIM = D_INNER + 2 * D_STATE
D_IN_PROJ = 2 * D_INNER + 2 * D_STATE + NHEADS
EPS = 1e-5

Q = 256
NCHUNKS = SEQLEN // Q


def _gab_kernel(x_ref, winz_ref, winx_ref, convw_ref, convb_ref, dtb_ref,
                alog_ref, drep_ref, normw_ref, wout_ref, e_ref, o_ref,
                xbc_ref, state_ref, xdt_ref):
    j = pl.program_id(1)

    xb = x_ref[0].astype(jnp.bfloat16)
    xbcdt = jnp.dot(xb, winx_ref[...], preferred_element_type=jnp.float32)

    @pl.when(j == 0)
    def _():
        xbc_ref[0:3, :] = jnp.zeros((3, CONV_DIM), jnp.float32)
        state_ref[...] = jnp.zeros_like(state_ref)

    xbc_ref[3:, :] = xbcdt[:, :CONV_DIM]
    xe = xbc_ref[...]
    conv = (convb_ref[0][None, :]
            + jax.lax.slice(xe, (0, 0), (Q, CONV_DIM)) * convw_ref[0][None, :]
            + jax.lax.slice(xe, (1, 0), (Q + 1, CONV_DIM)) * convw_ref[1][None, :]
            + jax.lax.slice(xe, (2, 0), (Q + 2, CONV_DIM)) * convw_ref[2][None, :]
            + jax.lax.slice(xe, (3, 0), (Q + 3, CONV_DIM)) * convw_ref[3][None, :])
    xbc = conv * jax.nn.sigmoid(conv)
    xbc_ref[0:3, :] = xbc_ref[Q:Q + 3, :]

    x_all = xbc[:, :D_INNER]
    Bm = xbc[:, D_INNER:D_INNER + D_STATE]
    Cm = xbc[:, D_INNER + D_STATE:]

    dt_raw = xbcdt[:, CONV_DIM:] + dtb_ref[0][None, :]
    dt = jnp.maximum(dt_raw, 0.0) + jnp.log1p(jnp.exp(-jnp.abs(dt_raw)))
    A = -jnp.exp(alog_ref[0])
    dtA = dt * A[None, :]

    rows = jax.lax.broadcasted_iota(jnp.int32, (Q, Q), 0)
    cols = jax.lax.broadcasted_iota(jnp.int32, (Q, Q), 1)
    tril = (rows >= cols).astype(jnp.float32)
    c = jnp.dot(tril, dtA, preferred_element_type=jnp.float32)
    cT = c.T
    c_end = c[Q - 1:Q, :]

    E = e_ref[...]
    dt_rep = jnp.dot(dt.astype(jnp.bfloat16), E,
                     preferred_element_type=jnp.float32)
    expc_rep = jnp.dot(jnp.exp(c).astype(jnp.bfloat16), E,
                       preferred_element_type=jnp.float32)
    dte_rep = jnp.dot(jnp.exp(c_end - c).astype(jnp.bfloat16), E,
                      preferred_element_type=jnp.float32)

    xdt_ref[...] = x_all * dt_rep

    G = jax.lax.dot_general(Cm, Bm, (((1,), (1,)), ((), ())),
                            preferred_element_type=jnp.float32)
    Gm = G * tril

    ys = []
    for h in range(NHEADS):
        seg = jnp.minimum(c[:, h:h + 1] - cT[h:h + 1, :], 0.0)
        Mh = Gm * jnp.exp(seg)
        ys.append(jnp.dot(Mh, xdt_ref[:, h * HEADDIM:(h + 1) * HEADDIM],
                          preferred_element_type=jnp.float32))
    y = jnp.concatenate(ys, axis=1)

    y = (y + jnp.dot(Cm, state_ref[...],
                     preferred_element_type=jnp.float32) * expc_rep
         + x_all * drep_ref[0][None, :])

    Snew = jax.lax.dot_general(Bm, xdt_ref[...] * dte_rep,
                               (((0,), (0,)), ((), ())),
                               preferred_element_type=jnp.float32)
    state_ref[...] = state_ref[...] * expc_rep[Q - 1:Q, :] + Snew

    z = jnp.dot(x_ref[0].astype(jnp.bfloat16), winz_ref[...],
                preferred_element_type=jnp.float32)
    y = y * (z * jax.nn.sigmoid(z))
    ssq = jnp.mean(y * y, axis=1, keepdims=True)
    y = y * jax.lax.rsqrt(ssq + EPS) * normw_ref[0][None, :]
    o_ref[0] = jnp.dot(y.astype(jnp.bfloat16), wout_ref[...],
                       preferred_element_type=jnp.float32)


def kernel(X, W_in, conv_w, conv_b, dt_bias, A_log, D, norm_w, W_out):
    win_t = W_in.T.astype(jnp.bfloat16)
    winz = win_t[:, :D_INNER]
    winx = win_t[:, D_INNER:]
    wout_t = W_out.T.astype(jnp.bfloat16)
    convw_t = conv_w.T
    d_rep = jnp.repeat(D, HEADDIM)[None, :]
    eye = jnp.eye(NHEADS, dtype=jnp.bfloat16)
    e_mat = jnp.repeat(eye, HEADDIM, axis=1)

    grid = (BATCH, NCHUNKS)
    return pl.pallas_call(
        _gab_kernel,
        out_shape=jax.ShapeDtypeStruct((BATCH, SEQLEN, D_MODEL), jnp.float32),
        grid=grid,
        in_specs=[
            pl.BlockSpec((1, Q, D_MODEL), lambda b, j: (b, j, 0)),
            pl.BlockSpec((D_MODEL, D_INNER), lambda b, j: (0, 0)),
            pl.BlockSpec((D_MODEL, D_IN_PROJ - D_INNER),
                         lambda b, j: (0, 0)),
            pl.BlockSpec((D_CONV, CONV_DIM), lambda b, j: (0, 0)),
            pl.BlockSpec((1, CONV_DIM), lambda b, j: (0, 0)),
            pl.BlockSpec((1, NHEADS), lambda b, j: (0, 0)),
            pl.BlockSpec((1, NHEADS), lambda b, j: (0, 0)),
            pl.BlockSpec((1, D_INNER), lambda b, j: (0, 0)),
            pl.BlockSpec((1, D_INNER), lambda b, j: (0, 0)),
            pl.BlockSpec((D_INNER, D_MODEL), lambda b, j: (0, 0)),
            pl.BlockSpec((NHEADS, D_INNER), lambda b, j: (0, 0)),
        ],
        out_specs=pl.BlockSpec((1, Q, D_MODEL), lambda b, j: (b, j, 0)),
        scratch_shapes=[
            pltpu.VMEM((Q + 3, CONV_DIM), jnp.float32),
            pltpu.VMEM((D_STATE, D_INNER), jnp.float32),
            pltpu.VMEM((Q, D_INNER), jnp.float32),
        ],
        compiler_params=pltpu.CompilerParams(
            dimension_semantics=("parallel", "arbitrary"),
            vmem_limit_bytes=56 * 1024 * 1024,
        ),
        name="gab_mamba2_fused",
    )(X, winz, winx, convw_t, conv_b[None, :], dt_bias[None, :],
      A_log[None, :], d_rep, norm_w[None, :], wout_t, e_mat)


# speedup vs baseline: 1.0257x; 1.0257x over previous
import jax
import jax.numpy as jnp
from jax.experimental import pallas as pl
from jax.experimental.pallas import tpu as pltpu

BATCH = 2
SEQLEN = 4096
D_MODEL = 1024
D_STATE = 128
HEADDIM = 64
D_CONV = 4
D_INNER = 2048
NHEADS = 32
CONV_DIM = D_INNER + 2 * D_STATE
D_IN_PROJ = 2 * D_INNER + 2 * D_STATE + NHEADS
EPS = 1e-5

Q = 256
NCHUNKS = SEQLEN // Q


def _gab_kernel(x_ref, win_ref, convw_ref, convb_ref, dtb_ref, alog_ref,
                drep_ref, normw_ref, wout_ref, e_ref, o_ref,
                zx_ref, xbc_ref, state_ref, xdt_ref):
    j = pl.program_id(1)

    xb = x_ref[0].astype(jnp.bfloat16)
    zx_ref[...] = jnp.dot(xb, win_ref[...],
                          preferred_element_type=jnp.float32)

    @pl.when(j == 0)
    def _():
        xbc_ref[0:3, :] = jnp.zeros((3, CONV_DIM), jnp.float32)
        state_ref[...] = jnp.zeros_like(state_ref)

    xbc_ref[3:, :] = zx_ref[:, D_INNER:D_INNER + CONV_DIM]
    xe = xbc_ref[...]
    conv = (convb_ref[0][None, :]
            + jax.lax.slice(xe, (0, 0), (Q, CONV_DIM)) * convw_ref[0][None, :]
            + jax.lax.slice(xe, (1, 0), (Q + 1, CONV_DIM)) * convw_ref[1][None, :]
            + jax.lax.slice(xe, (2, 0), (Q + 2, CONV_DIM)) * convw_ref[2][None, :]
            + jax.lax.slice(xe, (3, 0), (Q + 3, CONV_DIM)) * convw_ref[3][None, :])
    xbc = conv * jax.nn.sigmoid(conv)
    xbc_ref[0:3, :] = xbc_ref[Q:Q + 3, :]

    x_all = xbc[:, :D_INNER]
    Bm = xbc[:, D_INNER:D_INNER + D_STATE]
    Cm = xbc[:, D_INNER + D_STATE:]

    dt_raw = zx_ref[:, D_INNER + CONV_DIM:] + dtb_ref[0][None, :]
    dt = jnp.maximum(dt_raw, 0.0) + jnp.log1p(jnp.exp(-jnp.abs(dt_raw)))
    A = -jnp.exp(alog_ref[0])
    dtA = dt * A[None, :]

    rows = jax.lax.broadcasted_iota(jnp.int32, (Q, Q), 0)
    cols = jax.lax.broadcasted_iota(jnp.int32, (Q, Q), 1)
    tril = (rows >= cols).astype(jnp.float32)
    c = jnp.dot(tril, dtA, preferred_element_type=jnp.float32)
    cT = c.T
    c_end = c[Q - 1:Q, :]

    E = e_ref[...]
    dt_rep = jnp.dot(dt.astype(jnp.bfloat16), E,
                     preferred_element_type=jnp.float32)
    expc_rep = jnp.dot(jnp.exp(c).astype(jnp.bfloat16), E,
                       preferred_element_type=jnp.float32)
    dte_rep = jnp.dot(jnp.exp(c_end - c).astype(jnp.bfloat16), E,
                      preferred_element_type=jnp.float32)

    xdt_ref[...] = x_all * dt_rep

    G = jax.lax.dot_general(Cm, Bm, (((1,), (1,)), ((), ())),
                            preferred_element_type=jnp.float32)
    Gm = G * tril

    ys = []
    for h in range(NHEADS):
        seg = jnp.minimum(c[:, h:h + 1] - cT[h:h + 1, :], 0.0)
        Mh = Gm * jnp.exp(seg)
        ys.append(jnp.dot(Mh, xdt_ref[:, h * HEADDIM:(h + 1) * HEADDIM],
                          preferred_element_type=jnp.float32))
    y = jnp.concatenate(ys, axis=1)

    y = (y + jnp.dot(Cm, state_ref[...],
                     preferred_element_type=jnp.float32) * expc_rep
         + x_all * drep_ref[0][None, :])

    Snew = jax.lax.dot_general(Bm, xdt_ref[...] * dte_rep,
                               (((0,), (0,)), ((), ())),
                               preferred_element_type=jnp.float32)
    state_ref[...] = state_ref[...] * expc_rep[Q - 1:Q, :] + Snew

    z = zx_ref[:, 0:D_INNER]
    y = y * (z * jax.nn.sigmoid(z))
    ssq = jnp.mean(y * y, axis=1, keepdims=True)
    y = y * jax.lax.rsqrt(ssq + EPS) * normw_ref[0][None, :]
    o_ref[0] = jnp.dot(y.astype(jnp.bfloat16), wout_ref[...],
                       preferred_element_type=jnp.float32)


def kernel(X, W_in, conv_w, conv_b, dt_bias, A_log, D, norm_w, W_out):
    win_t = W_in.T.astype(jnp.bfloat16)
    wout_t = W_out.T.astype(jnp.bfloat16)
    convw_t = conv_w.T
    d_rep = jnp.repeat(D, HEADDIM)[None, :]
    eye = jnp.eye(NHEADS, dtype=jnp.bfloat16)
    e_mat = jnp.repeat(eye, HEADDIM, axis=1)

    grid = (BATCH, NCHUNKS)
    return pl.pallas_call(
        _gab_kernel,
        out_shape=jax.ShapeDtypeStruct((BATCH, SEQLEN, D_MODEL), jnp.float32),
        grid=grid,
        in_specs=[
            pl.BlockSpec((1, Q, D_MODEL), lambda b, j: (b, j, 0)),
            pl.BlockSpec((D_MODEL, D_IN_PROJ), lambda b, j: (0, 0)),
            pl.BlockSpec((D_CONV, CONV_DIM), lambda b, j: (0, 0)),
            pl.BlockSpec((1, CONV_DIM), lambda b, j: (0, 0)),
            pl.BlockSpec((1, NHEADS), lambda b, j: (0, 0)),
            pl.BlockSpec((1, NHEADS), lambda b, j: (0, 0)),
            pl.BlockSpec((1, D_INNER), lambda b, j: (0, 0)),
            pl.BlockSpec((1, D_INNER), lambda b, j: (0, 0)),
            pl.BlockSpec((D_INNER, D_MODEL), lambda b, j: (0, 0)),
            pl.BlockSpec((NHEADS, D_INNER), lambda b, j: (0, 0)),
        ],
        out_specs=pl.BlockSpec((1, Q, D_MODEL), lambda b, j: (b, j, 0)),
        scratch_shapes=[
            pltpu.VMEM((Q, D_IN_PROJ), jnp.float32),
            pltpu.VMEM((Q + 3, CONV_DIM), jnp.float32),
            pltpu.VMEM((D_STATE, D_INNER), jnp.float32),
            pltpu.VMEM((Q, D_INNER), jnp.float32),
        ],
        compiler_params=pltpu.CompilerParams(
            dimension_semantics=("parallel", "arbitrary"),
            vmem_limit_bytes=56 * 1024 * 1024,
        ),
        name="gab_mamba2_fused",
    )(X, win_t, convw_t, conv_b[None, :], dt_bias[None, :],
      A_log[None, :], d_rep, norm_w[None, :], wout_t, e_mat)


# speedup vs baseline: 1.0842x; 1.0571x over previous
import jax
import jax.numpy as jnp
import numpy as np
from jax.experimental import pallas as pl
from jax.experimental.pallas import tpu as pltpu

BATCH = 2
SEQLEN = 4096
D_MODEL = 1024
D_STATE = 128
HEADDIM = 64
D_CONV = 4
D_INNER = 2048
NHEADS = 32
CONV_DIM = D_INNER + 2 * D_STATE
D_IN_PROJ = 2 * D_INNER + 2 * D_STATE + NHEADS
EPS = 1e-5

Q = 256
NCHUNKS = SEQLEN // Q

_E_MAT = np.repeat(np.eye(NHEADS, dtype=np.float32), HEADDIM, axis=1)


def _gab_kernel(x_ref, win_ref, convw_ref, convb_ref, dtb_ref, alog_ref,
                drep_ref, normw_ref, wout_ref, e_ref, o_ref,
                zx_ref, xbc_ref, state_ref, xdt_ref):
    j = pl.program_id(1)

    xb = x_ref[0].astype(jnp.bfloat16)
    zx_ref[...] = jax.lax.dot_general(
        xb, win_ref[...], (((1,), (1,)), ((), ())),
        preferred_element_type=jnp.float32)

    @pl.when(j == 0)
    def _():
        xbc_ref[0:3, :] = jnp.zeros((3, CONV_DIM), jnp.float32)
        state_ref[...] = jnp.zeros_like(state_ref)

    xbc_ref[3:, :] = zx_ref[:, D_INNER:D_INNER + CONV_DIM]
    xe = xbc_ref[...]
    conv = (convb_ref[0][None, :]
            + jax.lax.slice(xe, (0, 0), (Q, CONV_DIM)) * convw_ref[0][None, :]
            + jax.lax.slice(xe, (1, 0), (Q + 1, CONV_DIM)) * convw_ref[1][None, :]
            + jax.lax.slice(xe, (2, 0), (Q + 2, CONV_DIM)) * convw_ref[2][None, :]
            + jax.lax.slice(xe, (3, 0), (Q + 3, CONV_DIM)) * convw_ref[3][None, :])
    xbc = conv * jax.nn.sigmoid(conv)
    xbc_ref[0:3, :] = xbc_ref[Q:Q + 3, :]

    x_all = xbc[:, :D_INNER]
    Bm = xbc[:, D_INNER:D_INNER + D_STATE]
    Cm = xbc[:, D_INNER + D_STATE:]

    dt_raw = zx_ref[:, D_INNER + CONV_DIM:] + dtb_ref[0][None, :]
    dt = jnp.maximum(dt_raw, 0.0) + jnp.log1p(jnp.exp(-jnp.abs(dt_raw)))
    A = -jnp.exp(alog_ref[0])
    dtA = dt * A[None, :]

    rows = jax.lax.broadcasted_iota(jnp.int32, (Q, Q), 0)
    cols = jax.lax.broadcasted_iota(jnp.int32, (Q, Q), 1)
    tril = (rows >= cols).astype(jnp.float32)
    c = jnp.dot(tril, dtA, preferred_element_type=jnp.float32)
    cT = c.T
    c_end = c[Q - 1:Q, :]

    E = e_ref[...]
    dt_rep = jnp.dot(dt.astype(jnp.bfloat16), E,
                     preferred_element_type=jnp.float32)
    expc_rep = jnp.dot(jnp.exp(c).astype(jnp.bfloat16), E,
                       preferred_element_type=jnp.float32)
    dte_rep = jnp.dot(jnp.exp(c_end - c).astype(jnp.bfloat16), E,
                      preferred_element_type=jnp.float32)

    xdt_ref[...] = x_all * dt_rep

    G = jax.lax.dot_general(Cm, Bm, (((1,), (1,)), ((), ())),
                            preferred_element_type=jnp.float32)
    Gm = G * tril

    ys = []
    for h in range(NHEADS):
        seg = jnp.minimum(c[:, h:h + 1] - cT[h:h + 1, :], 0.0)
        Mh = Gm * jnp.exp(seg)
        ys.append(jnp.dot(Mh, xdt_ref[:, h * HEADDIM:(h + 1) * HEADDIM],
                          preferred_element_type=jnp.float32))
    y = jnp.concatenate(ys, axis=1)

    y = (y + jnp.dot(Cm, state_ref[...],
                     preferred_element_type=jnp.float32) * expc_rep
         + x_all * drep_ref[0][None, :])

    Snew = jax.lax.dot_general(Bm, xdt_ref[...] * dte_rep,
                               (((0,), (0,)), ((), ())),
                               preferred_element_type=jnp.float32)
    state_ref[...] = state_ref[...] * expc_rep[Q - 1:Q, :] + Snew

    z = zx_ref[:, 0:D_INNER]
    y = y * (z * jax.nn.sigmoid(z))
    ssq = jnp.mean(y * y, axis=1, keepdims=True)
    y = y * jax.lax.rsqrt(ssq + EPS) * normw_ref[0][None, :]
    o_ref[0] = jax.lax.dot_general(
        y.astype(jnp.bfloat16), wout_ref[...], (((1,), (1,)), ((), ())),
        preferred_element_type=jnp.float32)


def kernel(X, W_in, conv_w, conv_b, dt_bias, A_log, D, norm_w, W_out):
    win_b = W_in.astype(jnp.bfloat16)
    wout_b = W_out.astype(jnp.bfloat16)
    convw_t = conv_w.T
    d_rep = jnp.repeat(D, HEADDIM)[None, :]
    e_mat = jnp.asarray(_E_MAT, jnp.bfloat16)

    grid = (BATCH, NCHUNKS)
    return pl.pallas_call(
        _gab_kernel,
        out_shape=jax.ShapeDtypeStruct((BATCH, SEQLEN, D_MODEL), jnp.float32),
        grid=grid,
        in_specs=[
            pl.BlockSpec((1, Q, D_MODEL), lambda b, j: (b, j, 0)),
            pl.BlockSpec((D_IN_PROJ, D_MODEL), lambda b, j: (0, 0)),
            pl.BlockSpec((D_CONV, CONV_DIM), lambda b, j: (0, 0)),
            pl.BlockSpec((1, CONV_DIM), lambda b, j: (0, 0)),
            pl.BlockSpec((1, NHEADS), lambda b, j: (0, 0)),
            pl.BlockSpec((1, NHEADS), lambda b, j: (0, 0)),
            pl.BlockSpec((1, D_INNER), lambda b, j: (0, 0)),
            pl.BlockSpec((1, D_INNER), lambda b, j: (0, 0)),
            pl.BlockSpec((D_MODEL, D_INNER), lambda b, j: (0, 0)),
            pl.BlockSpec((NHEADS, D_INNER), lambda b, j: (0, 0)),
        ],
        out_specs=pl.BlockSpec((1, Q, D_MODEL), lambda b, j: (b, j, 0)),
        scratch_shapes=[
            pltpu.VMEM((Q, D_IN_PROJ), jnp.float32),
            pltpu.VMEM((Q + 3, CONV_DIM), jnp.float32),
            pltpu.VMEM((D_STATE, D_INNER), jnp.float32),
            pltpu.VMEM((Q, D_INNER), jnp.float32),
        ],
        compiler_params=pltpu.CompilerParams(
            dimension_semantics=("parallel", "arbitrary"),
            vmem_limit_bytes=56 * 1024 * 1024,
        ),
        name="gab_mamba2_fused",
    )(X, win_b, convw_t, conv_b[None, :], dt_bias[None, :],
      A_log[None, :], d_rep, norm_w[None, :], wout_b, e_mat)


# speedup vs baseline: 1.1003x; 1.0148x over previous
import jax
import jax.numpy as jnp
import numpy as np
from jax.experimental import pallas as pl
from jax.experimental.pallas import tpu as pltpu

BATCH = 2
SEQLEN = 4096
D_MODEL = 1024
D_STATE = 128
HEADDIM = 64
D_CONV = 4
D_INNER = 2048
NHEADS = 32
CONV_DIM = D_INNER + 2 * D_STATE
D_IN_PROJ = 2 * D_INNER + 2 * D_STATE + NHEADS
EPS = 1e-5

Q = 256
NCHUNKS = SEQLEN // Q

_E_MAT = np.repeat(np.eye(NHEADS, dtype=np.float32), HEADDIM, axis=1)


def _gab_kernel(x_ref, win_ref, convw_ref, convb_ref, dtb_ref, alog_ref,
                drep_ref, normw_ref, wout_ref, e_ref, o_ref,
                zx_ref, xbc_ref, state_ref, xdt_ref):
    j = pl.program_id(1)

    xb = x_ref[0].astype(jnp.bfloat16)
    zx_ref[...] = jax.lax.dot_general(
        xb, win_ref[...], (((1,), (1,)), ((), ())),
        preferred_element_type=jnp.float32)

    @pl.when(j == 0)
    def _():
        xbc_ref[0:3, :] = jnp.zeros((3, CONV_DIM), jnp.float32)
        state_ref[...] = jnp.zeros_like(state_ref)

    xbc_ref[3:, :] = zx_ref[:, D_INNER:D_INNER + CONV_DIM]
    xe = xbc_ref[...]
    conv = (convb_ref[0][None, :]
            + jax.lax.slice(xe, (0, 0), (Q, CONV_DIM)) * convw_ref[0][None, :]
            + jax.lax.slice(xe, (1, 0), (Q + 1, CONV_DIM)) * convw_ref[1][None, :]
            + jax.lax.slice(xe, (2, 0), (Q + 2, CONV_DIM)) * convw_ref[2][None, :]
            + jax.lax.slice(xe, (3, 0), (Q + 3, CONV_DIM)) * convw_ref[3][None, :])
    xbc = conv * jax.nn.sigmoid(conv)
    xbc_ref[0:3, :] = xbc_ref[Q:Q + 3, :]

    x_all = xbc[:, :D_INNER]
    Bm = xbc[:, D_INNER:D_INNER + D_STATE]
    Cm = xbc[:, D_INNER + D_STATE:]

    dt_raw = zx_ref[:, D_INNER + CONV_DIM:] + dtb_ref[0][None, :]
    dt = jnp.maximum(dt_raw, 0.0) + jnp.log1p(jnp.exp(-jnp.abs(dt_raw)))
    A = -jnp.exp(alog_ref[0])
    dtA = dt * A[None, :]

    rows = jax.lax.broadcasted_iota(jnp.int32, (Q, Q), 0)
    cols = jax.lax.broadcasted_iota(jnp.int32, (Q, Q), 1)
    tril = (rows >= cols).astype(jnp.float32)
    c = jnp.dot(tril, dtA * np.float32(1.4426950408889634),
                preferred_element_type=jnp.float32)
    cT = c.T
    c_end = c[Q - 1:Q, :]

    E = e_ref[...]
    dt_rep = jnp.dot(dt.astype(jnp.bfloat16), E,
                     preferred_element_type=jnp.float32)
    expc_rep = jnp.dot(jnp.exp2(c).astype(jnp.bfloat16), E,
                       preferred_element_type=jnp.float32)
    dte_rep = jnp.dot(jnp.exp2(c_end - c).astype(jnp.bfloat16), E,
                      preferred_element_type=jnp.float32)

    xdt_ref[...] = x_all * dt_rep

    G = jax.lax.dot_general(Cm, Bm, (((1,), (1,)), ((), ())),
                            preferred_element_type=jnp.float32)
    Gm = G * tril

    ys = []
    for h in range(NHEADS):
        seg = jnp.minimum(c[:, h:h + 1] - cT[h:h + 1, :], 0.0)
        Mh = Gm * jnp.exp2(seg)
        ys.append(jnp.dot(Mh, xdt_ref[:, h * HEADDIM:(h + 1) * HEADDIM],
                          preferred_element_type=jnp.float32))
    y = jnp.concatenate(ys, axis=1)

    y = (y + jnp.dot(Cm, state_ref[...],
                     preferred_element_type=jnp.float32) * expc_rep
         + x_all * drep_ref[0][None, :])

    Snew = jax.lax.dot_general(Bm, xdt_ref[...] * dte_rep,
                               (((0,), (0,)), ((), ())),
                               preferred_element_type=jnp.float32)
    state_ref[...] = state_ref[...] * expc_rep[Q - 1:Q, :] + Snew

    z = zx_ref[:, 0:D_INNER]
    y = y * (z * jax.nn.sigmoid(z))
    ssq = jnp.mean(y * y, axis=1, keepdims=True)
    y = y * jax.lax.rsqrt(ssq + EPS) * normw_ref[0][None, :]
    o_ref[0] = jax.lax.dot_general(
        y.astype(jnp.bfloat16), wout_ref[...], (((1,), (1,)), ((), ())),
        preferred_element_type=jnp.float32)


def kernel(X, W_in, conv_w, conv_b, dt_bias, A_log, D, norm_w, W_out):
    win_b = W_in.astype(jnp.bfloat16)
    wout_b = W_out.astype(jnp.bfloat16)
    convw_t = conv_w.T
    d_rep = jnp.repeat(D, HEADDIM)[None, :]
    e_mat = jnp.asarray(_E_MAT, jnp.bfloat16)

    grid = (BATCH, NCHUNKS)
    return pl.pallas_call(
        _gab_kernel,
        out_shape=jax.ShapeDtypeStruct((BATCH, SEQLEN, D_MODEL), jnp.float32),
        grid=grid,
        in_specs=[
            pl.BlockSpec((1, Q, D_MODEL), lambda b, j: (b, j, 0)),
            pl.BlockSpec((D_IN_PROJ, D_MODEL), lambda b, j: (0, 0)),
            pl.BlockSpec((D_CONV, CONV_DIM), lambda b, j: (0, 0)),
            pl.BlockSpec((1, CONV_DIM), lambda b, j: (0, 0)),
            pl.BlockSpec((1, NHEADS), lambda b, j: (0, 0)),
            pl.BlockSpec((1, NHEADS), lambda b, j: (0, 0)),
            pl.BlockSpec((1, D_INNER), lambda b, j: (0, 0)),
            pl.BlockSpec((1, D_INNER), lambda b, j: (0, 0)),
            pl.BlockSpec((D_MODEL, D_INNER), lambda b, j: (0, 0)),
            pl.BlockSpec((NHEADS, D_INNER), lambda b, j: (0, 0)),
        ],
        out_specs=pl.BlockSpec((1, Q, D_MODEL), lambda b, j: (b, j, 0)),
        scratch_shapes=[
            pltpu.VMEM((Q, D_IN_PROJ), jnp.float32),
            pltpu.VMEM((Q + 3, CONV_DIM), jnp.float32),
            pltpu.VMEM((D_STATE, D_INNER), jnp.float32),
            pltpu.VMEM((Q, D_INNER), jnp.float32),
        ],
        compiler_params=pltpu.CompilerParams(
            dimension_semantics=("parallel", "arbitrary"),
            vmem_limit_bytes=56 * 1024 * 1024,
        ),
        name="gab_mamba2_fused",
    )(X, win_b, convw_t, conv_b[None, :], dt_bias[None, :],
      A_log[None, :], d_rep, norm_w[None, :], wout_b, e_mat)
